# Initial kernel scaffold; baseline (speedup 1.0000x reference)
import jax
import jax.numpy as jnp
from jax.experimental import pallas as pl


def kernel(input_ids, tok_emb, hash_emb, enc, dec, final_norm, Wout):
    raise NotImplementedError("write your pallas kernel here")


# speedup vs baseline: 3.3456x; 3.3456x over previous
import functools

import numpy as np
import jax
import jax.numpy as jnp
from jax import lax
from jax.experimental import pallas as pl
from jax.experimental.pallas import tpu as pltpu
from jax.experimental.pallas import tpu_sc as plsc

_PRIMES = (2654435761, 805306457, 2166136261)
_HV = 100000
_B, _S, _D, _DFF = 4, 2048, 768, 2048
_HALF = _D // 2
_CW = 512
_NCH = _S // _CW
_FC = 1024
_NW = 32

_INTERPRET = False


def _rms(x, w):
    return x * lax.rsqrt(jnp.mean(x * x, axis=-1, keepdims=True) + 1e-5) * w


def _qkv_body(x_ref, nw_ref, wq_ref, wk_ref, wv_ref, cos_ref, sin_ref,
              q_ref, k_ref, v_ref):
    x = x_ref[0]
    nx = _rms(x, nw_ref[0])
    cos = cos_ref[...]
    sin = sin_ref[...]

    def rope(t):
        t1 = t[:, :_HALF]
        t2 = t[:, _HALF:]
        return jnp.concatenate([t1 * cos - t2 * sin, t1 * sin + t2 * cos],
                               axis=-1)

    q_ref[0] = rope(jnp.dot(nx, wq_ref[...], preferred_element_type=jnp.float32))
    k_ref[0] = rope(jnp.dot(nx, wk_ref[...], preferred_element_type=jnp.float32))
    v_ref[0] = jnp.dot(nx, wv_ref[...], preferred_element_type=jnp.float32)


def _qkv_call(x, nw, wq, wk, wv, cos, sin):
    f = jax.ShapeDtypeStruct((_B, _S, _D), jnp.float32)
    return pl.pallas_call(
        _qkv_body,
        grid=(_B, _NCH),
        in_specs=[
            pl.BlockSpec((1, _CW, _D), lambda b, c: (b, c, 0)),
            pl.BlockSpec((1, _D), lambda b, c: (0, 0)),
            pl.BlockSpec((_D, _D), lambda b, c: (0, 0)),
            pl.BlockSpec((_D, _D), lambda b, c: (0, 0)),
            pl.BlockSpec((_D, _D), lambda b, c: (0, 0)),
            pl.BlockSpec((_CW, _HALF), lambda b, c: (c, 0)),
            pl.BlockSpec((_CW, _HALF), lambda b, c: (c, 0)),
        ],
        out_specs=[pl.BlockSpec((1, _CW, _D), lambda b, c: (b, c, 0))] * 3,
        out_shape=[f, f, f],
        interpret=_INTERPRET,
    )(x, nw, wq, wk, wv, cos, sin)


def _attn_body(q_ref, kp_ref, kc_ref, vp_ref, vc_ref, o_ref):
    c = pl.program_id(1)
    scale = np.float32(1.0 / np.sqrt(_D))
    q = q_ref[0] * scale
    sp = lax.dot_general(q, kp_ref[0], (((1,), (1,)), ((), ())),
                         preferred_element_type=jnp.float32)
    sc = lax.dot_general(q, kc_ref[0], (((1,), (1,)), ((), ())),
                         preferred_element_type=jnp.float32)
    i = lax.broadcasted_iota(jnp.int32, (_CW, _CW), 0)
    j = lax.broadcasted_iota(jnp.int32, (_CW, _CW), 1)
    mp = jnp.logical_and(i < j, c > 0)
    mc = j <= i
    sp = jnp.where(mp, sp, -1e30)
    sc = jnp.where(mc, sc, -1e30)
    m = jnp.maximum(jnp.max(sp, axis=1, keepdims=True),
                    jnp.max(sc, axis=1, keepdims=True))
    ep = jnp.exp(sp - m)
    ec = jnp.exp(sc - m)
    den = jnp.sum(ep, axis=1, keepdims=True) + jnp.sum(ec, axis=1, keepdims=True)
    o = (jnp.dot(ep, vp_ref[0], preferred_element_type=jnp.float32) +
         jnp.dot(ec, vc_ref[0], preferred_element_type=jnp.float32))
    o_ref[0] = o / den


def _attn_call(q, k, v):
    blk = lambda b, c: (b, c, 0)
    blkp = lambda b, c: (b, jnp.maximum(c - 1, 0), 0)
    return pl.pallas_call(
        _attn_body,
        grid=(_B, _NCH),
        in_specs=[
            pl.BlockSpec((1, _CW, _D), blk),
            pl.BlockSpec((1, _CW, _D), blkp),
            pl.BlockSpec((1, _CW, _D), blk),
            pl.BlockSpec((1, _CW, _D), blkp),
            pl.BlockSpec((1, _CW, _D), blk),
        ],
        out_specs=pl.BlockSpec((1, _CW, _D), blk),
        out_shape=jax.ShapeDtypeStruct((_B, _S, _D), jnp.float32),
        interpret=_INTERPRET,
    )(q, k, k, v, v)


def _oproj_body(x_ref, a_ref, wo_ref, fw_ref, h_ref, y_ref):
    h = x_ref[0] + jnp.dot(a_ref[0], wo_ref[...],
                           preferred_element_type=jnp.float32)
    h_ref[0] = h
    y_ref[0] = _rms(h, fw_ref[0])


def _oproj_call(x, a, wo, fw):
    f = jax.ShapeDtypeStruct((_B, _S, _D), jnp.float32)
    return pl.pallas_call(
        _oproj_body,
        grid=(_B, _NCH),
        in_specs=[
            pl.BlockSpec((1, _CW, _D), lambda b, c: (b, c, 0)),
            pl.BlockSpec((1, _CW, _D), lambda b, c: (b, c, 0)),
            pl.BlockSpec((_D, _D), lambda b, c: (0, 0)),
            pl.BlockSpec((1, _D), lambda b, c: (0, 0)),
        ],
        out_specs=[pl.BlockSpec((1, _CW, _D), lambda b, c: (b, c, 0))] * 2,
        out_shape=[f, f],
        interpret=_INTERPRET,
    )(x, a, wo, fw)


def _ffn_body(y_ref, h_ref, w1_ref, w3_ref, w2_ref, o_ref):
    c = pl.program_id(2)
    y = y_ref[0]
    u = jnp.dot(y, w1_ref[...], preferred_element_type=jnp.float32)
    g = u * (1.0 / (1.0 + jnp.exp(-u)))
    t = g * jnp.dot(y, w3_ref[...], preferred_element_type=jnp.float32)
    z = jnp.dot(t, w2_ref[...], preferred_element_type=jnp.float32)

    @pl.when(c == 0)
    def _():
        o_ref[0] = h_ref[0] + z

    @pl.when(c != 0)
    def _():
        o_ref[0] = o_ref[0] + z


def _ffn_call(y, h, w1, w3, w2):
    return pl.pallas_call(
        _ffn_body,
        grid=(_B, _NCH, _DFF // _FC),
        in_specs=[
            pl.BlockSpec((1, _CW, _D), lambda b, s, c: (b, s, 0)),
            pl.BlockSpec((1, _CW, _D), lambda b, s, c: (b, s, 0)),
            pl.BlockSpec((_D, _FC), lambda b, s, c: (0, c)),
            pl.BlockSpec((_D, _FC), lambda b, s, c: (0, c)),
            pl.BlockSpec((_FC, _D), lambda b, s, c: (c, 0)),
        ],
        out_specs=pl.BlockSpec((1, _CW, _D), lambda b, s, c: (b, s, 0)),
        out_shape=jax.ShapeDtypeStruct((_B, _S, _D), jnp.float32),
        interpret=_INTERPRET,
    )(y, h, w1, w3, w2)


def _scan_body(h_ref, ids_ref, f_ref, g_ref):
    b = pl.program_id(0)
    ids = ids_ref[0]
    bnd = jnp.concatenate(
        [jnp.zeros((1, 1), jnp.int32), (ids[:-1] == 32).astype(jnp.int32)],
        axis=0)
    pid = bnd
    d = 1
    while d < _S:
        pid = pid + jnp.concatenate(
            [jnp.zeros((d, 1), jnp.int32), pid[:_S - d]], axis=0)
        d *= 2
    e = lax.broadcasted_iota(jnp.int32, (_S, 1), 0)
    d = 1
    while d < _S:
        pid_dn = jnp.concatenate([pid[d:], jnp.full((d, 1), -1, jnp.int32)],
                                 axis=0)
        e_dn = jnp.concatenate([e[d:], jnp.zeros((d, 1), jnp.int32)], axis=0)
        e = jnp.where(pid_dn == pid, jnp.maximum(e, e_dn), e)
        d *= 2
    f = h_ref[0]
    d = 1
    while d < _S:
        pid_up = jnp.concatenate([jnp.full((d, 1), -1, jnp.int32),
                                  pid[:_S - d]], axis=0)
        f_up = jnp.concatenate([jnp.full((d, _D), -3.0e38, jnp.float32),
                                f[:_S - d]], axis=0)
        f = jnp.where(pid_up == pid, jnp.maximum(f, f_up), f)
        d *= 2
    f_ref[0] = f
    g_ref[0] = e + b * _S


def _scan_call(h, ids3):
    return pl.pallas_call(
        _scan_body,
        grid=(_B,),
        in_specs=[
            pl.BlockSpec((1, _S, _D), lambda b: (b, 0, 0)),
            pl.BlockSpec((1, _S, 1), lambda b: (b, 0, 0)),
        ],
        out_specs=[
            pl.BlockSpec((1, _S, _D), lambda b: (b, 0, 0)),
            pl.BlockSpec((1, _S, 1), lambda b: (b, 0, 0)),
        ],
        out_shape=[
            jax.ShapeDtypeStruct((_B, _S, _D), jnp.float32),
            jax.ShapeDtypeStruct((_B, _S, 1), jnp.int32),
        ],
        interpret=_INTERPRET,
    )(h, ids3)


def _final_body(x_ref, fn_ref, w_ref, o_ref):
    o_ref[0] = jnp.dot(_rms(x_ref[0], fn_ref[0]), w_ref[...],
                       preferred_element_type=jnp.float32)


def _final_call(x, fn, wp):
    n = wp.shape[1]
    return pl.pallas_call(
        _final_body,
        grid=(_B, _NCH),
        in_specs=[
            pl.BlockSpec((1, _CW, _D), lambda b, c: (b, c, 0)),
            pl.BlockSpec((1, _D), lambda b, c: (0, 0)),
            pl.BlockSpec((_D, n), lambda b, c: (0, 0)),
        ],
        out_specs=pl.BlockSpec((1, _CW, n), lambda b, c: (b, c, 0)),
        out_shape=jax.ShapeDtypeStruct((_B, _S, n), jnp.float32),
        interpret=_INTERPRET,
    )(x, fn, wp)


def _sc_mesh():
    return plsc.VectorSubcoreMesh(core_axis_name="c", subcore_axis_name="s",
                                  num_cores=2, num_subcores=16)


def _sc_embed(ids_flat, tok_emb, hash_flat):
    bs = ids_flat.shape[0]
    tpb = bs // _NW
    nch = tpb // 32

    @functools.partial(
        pl.kernel,
        mesh=_sc_mesh(),
        out_type=jax.ShapeDtypeStruct((bs, _D), jnp.float32),
        scratch_types=[
            pltpu.VMEM((tpb + 16,), jnp.int32),
            pltpu.VMEM((96,), jnp.int32),
            pltpu.VMEM((32,), jnp.int32),
            pltpu.VMEM((96, _D), jnp.float32),
            pltpu.VMEM((32, _D), jnp.float32),
            pltpu.SemaphoreType.DMA,
            pltpu.SemaphoreType.DMA,
        ],
    )
    def k(ids_hbm, tok_hbm, hash_hbm, out_hbm,
          ids_v, hidx_v, tidx_v, hrows_v, acc_v, sem_h, sem_t):
        wid = lax.axis_index("s") * 2 + lax.axis_index("c")
        base = wid * tpb
        pltpu.sync_copy(ids_hbm.at[pl.ds(base, tpb)], ids_v.at[pl.ds(16, tpb)])
        row_start = (base % _S) == 0

        @pl.when(row_start)
        def _():
            ids_v[pl.ds(0, 16)] = jnp.zeros((16,), jnp.int32)

        @pl.when(jnp.logical_not(row_start))
        def _():
            pltpu.sync_copy(ids_hbm.at[pl.ds(base - 16, 16)],
                            ids_v.at[pl.ds(0, 16)])

        for ci in range(nch):
            for v in range(2):
                off = 16 + ci * 32 + v * 16
                a3 = ids_v[pl.ds(off, 16)]
                u3 = a3.astype(jnp.uint32)
                u2 = ids_v[pl.ds(off - 1, 16)].astype(jnp.uint32)
                u1 = ids_v[pl.ds(off - 2, 16)].astype(jnp.uint32)
                u0 = ids_v[pl.ds(off - 3, 16)].astype(jnp.uint32)
                for f in range(3):
                    p = jnp.uint32(_PRIMES[f])
                    hsh = ((u0 * p + u1) * p + u2) * p + u3
                    hidx_v[pl.ds(f * 32 + v * 16, 16)] = (
                        (hsh % jnp.uint32(_HV)).astype(jnp.int32)
                        + jnp.int32(f * _HV))
                tidx_v[pl.ds(v * 16, 16)] = a3
            cph = pltpu.async_copy(hash_hbm.at[hidx_v], hrows_v, sem_h)
            cpt = pltpu.async_copy(tok_hbm.at[tidx_v], acc_v, sem_t)
            cph.wait()
            cpt.wait()

            def add_body(r, carry):
                for cb in range(_D // 16):
                    sl = pl.ds(cb * 16, 16)
                    acc_v[r, sl] = (acc_v[r, sl] + hrows_v[r, sl]
                                    + hrows_v[r + 32, sl] + hrows_v[r + 64, sl])
                return carry

            lax.fori_loop(0, 32, add_body, 0)
            pltpu.sync_copy(acc_v, out_hbm.at[pl.ds(base + ci * 32, 32)])

    return k(ids_flat, tok_emb, hash_flat)


def _sc_gather_add(f_flat, gidx_flat, emb_flat):
    bs, d = f_flat.shape
    tpb = bs // _NW
    ch = 64
    nch = tpb // ch

    @functools.partial(
        pl.kernel,
        mesh=_sc_mesh(),
        out_type=jax.ShapeDtypeStruct((bs, d), jnp.float32),
        scratch_types=[
            pltpu.VMEM((ch,), jnp.int32),
            pltpu.VMEM((ch, d), jnp.float32),
            pltpu.VMEM((ch, d), jnp.float32),
            pltpu.SemaphoreType.DMA,
        ],
    )
    def k(f_hbm, gidx_hbm, emb_hbm, out_hbm, idx_v, grows_v, erows_v, sem):
        wid = lax.axis_index("s") * 2 + lax.axis_index("c")
        base = wid * tpb
        for ci in range(nch):
            o = base + ci * ch
            pltpu.sync_copy(gidx_hbm.at[pl.ds(o, ch)], idx_v)
            cp = pltpu.async_copy(f_hbm.at[idx_v], grows_v, sem)
            pltpu.sync_copy(emb_hbm.at[pl.ds(o, ch)], erows_v)
            cp.wait()

            def add_body(r, carry):
                for cb in range(d // 16):
                    sl = pl.ds(cb * 16, 16)
                    grows_v[r, sl] = grows_v[r, sl] + erows_v[r, sl]
                return carry

            lax.fori_loop(0, ch, add_body, 0)
            pltpu.sync_copy(grows_v, out_hbm.at[pl.ds(o, ch)])

    return k(f_flat, gidx_flat, emb_flat)


def _tc_block(x, p, cos, sin):
    q, k, v = _qkv_call(x, p["attn_norm"].reshape(1, _D),
                        p["Wq"], p["Wk"], p["Wv"], cos, sin)
    a = _attn_call(q, k, v)
    h, y = _oproj_call(x, a, p["Wo"], p["ffn_norm"].reshape(1, _D))
    return _ffn_call(y, h, p["w1"], p["w3"], p["w2"])


def kernel(input_ids, tok_emb, hash_emb, enc, dec, final_norm, Wout):
    ids_flat = input_ids.reshape(-1)
    hash_flat = hash_emb.reshape(-1, _D)
    embeds = _sc_embed(ids_flat, tok_emb, hash_flat)
    emb3 = embeds.reshape(_B, _S, _D)

    inv = 1.0 / (10000.0 ** (jnp.arange(_HALF, dtype=jnp.float32) / _HALF))
    ang = jnp.arange(_S, dtype=jnp.float32)[:, None] * inv[None, :]
    cos = jnp.cos(ang)
    sin = jnp.sin(ang)

    h_enc = _tc_block(emb3, enc, cos, sin)
    f, gidx = _scan_call(h_enc, input_ids.reshape(_B, _S, 1))
    dec_in = _sc_gather_add(f.reshape(_B * _S, _D), gidx.reshape(-1), embeds)
    h_dec = _tc_block(dec_in.reshape(_B, _S, _D), dec, cos, sin)

    nv = Wout.shape[1]
    wp = jnp.pad(Wout, ((0, 0), (0, 384 - nv)))
    logits = _final_call(h_dec, final_norm.reshape(1, _D), wp)
    return logits[..., :nv]
